# baseline (speedup 1.0000x reference)
import jax
import jax.numpy as jnp
import numpy as np
from jax import lax
from jax.experimental import pallas as pl
from jax.experimental.pallas import tpu as pltpu
from jax.experimental.pallas import tpu_sc as plsc

_G = 32
_NBUF = 4
_LAG = 2

_NC = 2
_NS = 16


def _make_fast_body(nrows):
    nchunks = nrows // _G

    def body(src, fast, bufs, in_sems, out_sems):
        def in_copy(g):
            b = g % _NBUF
            return pltpu.make_async_copy(
                src.at[pl.ds(g * _G, _G)], bufs.at[b], in_sems.at[b])

        def out_copy(g):
            b = g % _NBUF
            return pltpu.make_async_copy(
                bufs.at[b], fast.at[pl.ds(g * _G, _G)], out_sems.at[b])

        for g in range(min(_NBUF, nchunks)):
            in_copy(g).start()
        for g in range(nchunks):
            in_copy(g).wait()
            out_copy(g).start()
            gl = g - _LAG
            if gl >= 0:
                out_copy(gl).wait()
                if gl + _NBUF < nchunks:
                    in_copy(gl + _NBUF).start()
        for g in range(max(0, nchunks - _LAG), nchunks):
            out_copy(g).wait()

    return body


def _make_sc_body(assign):
    def body(src, out, buf, isems, osems):
        wid = lax.axis_index("s") * _NC + lax.axis_index("c")
        for w, tasks in enumerate(assign):
            if not tasks:
                continue

            @pl.when(wid == w)
            def _(tasks=tasks):
                ins = [
                    pltpu.make_async_copy(src.at[s], buf.at[i], isems.at[i])
                    for i, (d, s) in enumerate(tasks)
                ]
                outs = [
                    pltpu.make_async_copy(buf.at[i], out.at[d], osems.at[i])
                    for i, (d, s) in enumerate(tasks)
                ]
                for cp in ins:
                    cp.start()
                for i in range(len(tasks)):
                    ins[i].wait()
                    outs[i].start()
                for cp in outs:
                    cp.wait()

    return body


def _linspace_idx(stop, num):
    div = num - 1
    step = np.arange(div, dtype=np.float32) / np.float32(div)
    out = (np.float32(0.0) * (np.float32(1.0) - step)
           + np.float32(stop) * step)
    out = np.concatenate([out, np.array([stop], dtype=np.float32)])
    return out.astype(np.int32)


def kernel(frames):
    C, T, H, W = frames.shape
    alpha = 4
    n = T // alpha
    idx = _linspace_idx(float(T - 1), n)

    nrows = C * T
    nslow = C * n
    nw = _NC * _NS
    tasks = [(c * n + j, c * T + int(t))
             for c in range(C) for j, t in enumerate(idx.tolist())]
    assign = [[] for _ in range(nw)]
    for j, task in enumerate(tasks):
        assign[j % nw].append(task)
    nbuf = max(len(a) for a in assign)

    flat = frames.reshape(nrows, H, W)

    mesh = plsc.VectorSubcoreMesh(core_axis_name="c", subcore_axis_name="s")
    slow_flat = pl.kernel(
        _make_sc_body(assign),
        out_type=jax.ShapeDtypeStruct((nslow, H, W), jnp.float32),
        mesh=mesh,
        scratch_types=[
            pltpu.VMEM((nbuf, H, W), jnp.float32),
            pltpu.SemaphoreType.DMA((nbuf,)),
            pltpu.SemaphoreType.DMA((nbuf,)),
        ],
    )(flat)

    hbm = pl.BlockSpec(memory_space=pltpu.MemorySpace.HBM)
    fast_flat = pl.pallas_call(
        _make_fast_body(nrows),
        in_specs=[hbm],
        out_specs=hbm,
        out_shape=jax.ShapeDtypeStruct((nrows, H, W), jnp.float32),
        scratch_shapes=[
            pltpu.VMEM((_NBUF, _G, H, W), jnp.float32),
            pltpu.SemaphoreType.DMA((_NBUF,)),
            pltpu.SemaphoreType.DMA((_NBUF,)),
        ],
    )(flat)
    return (slow_flat.reshape(C, n, H, W), fast_flat.reshape(C, T, H, W))


# speedup vs baseline: 1.6648x; 1.6648x over previous
import jax
import jax.numpy as jnp
import numpy as np
from jax.experimental import pallas as pl
from jax.experimental.pallas import tpu as pltpu

_G = 32
_NBUF = 6
_LAG = 3


def _make_body(nrows, slow_map):
    nchunks = nrows // _G

    def body(src, fast, slow, bufs, in_sems, out_sems):
        def in_copy(g):
            b = g % _NBUF
            return pltpu.make_async_copy(
                src.at[pl.ds(g * _G, _G)], bufs.at[b], in_sems.at[b])

        def out_copies(g):
            b = g % _NBUF
            cps = [pltpu.make_async_copy(
                bufs.at[b], fast.at[pl.ds(g * _G, _G)], out_sems.at[b])]
            for k, r in slow_map[g]:
                cps.append(pltpu.make_async_copy(
                    bufs.at[b].at[r], slow.at[k], out_sems.at[b]))
            return cps

        for g in range(min(_NBUF, nchunks)):
            in_copy(g).start()
        for g in range(nchunks):
            in_copy(g).wait()
            for cp in out_copies(g):
                cp.start()
            gl = g - _LAG
            if gl >= 0:
                for cp in out_copies(gl):
                    cp.wait()
                if gl + _NBUF < nchunks:
                    in_copy(gl + _NBUF).start()
        for g in range(max(0, nchunks - _LAG), nchunks):
            for cp in out_copies(g):
                cp.wait()

    return body


def _linspace_idx(stop, num):
    div = num - 1
    step = np.arange(div, dtype=np.float32) / np.float32(div)
    out = (np.float32(0.0) * (np.float32(1.0) - step)
           + np.float32(stop) * step)
    out = np.concatenate([out, np.array([stop], dtype=np.float32)])
    return out.astype(np.int32)


def kernel(frames):
    C, T, H, W = frames.shape
    alpha = 4
    n = T // alpha
    idx = _linspace_idx(float(T - 1), n)

    nrows = C * T
    slow_map = {g: [] for g in range(nrows // _G)}
    for c in range(C):
        for j, t in enumerate(idx.tolist()):
            r = c * T + t
            slow_map[r // _G].append((c * n + j, r % _G))

    flat = frames.reshape(nrows, H, W)
    hbm = pl.BlockSpec(memory_space=pltpu.MemorySpace.HBM)
    fast_flat, slow_flat = pl.pallas_call(
        _make_body(nrows, slow_map),
        in_specs=[hbm],
        out_specs=[hbm, hbm],
        out_shape=[
            jax.ShapeDtypeStruct((nrows, H, W), jnp.float32),
            jax.ShapeDtypeStruct((C * n, H, W), jnp.float32),
        ],
        scratch_shapes=[
            pltpu.VMEM((_NBUF, _G, H, W), jnp.float32),
            pltpu.SemaphoreType.DMA((_NBUF,)),
            pltpu.SemaphoreType.DMA((_NBUF,)),
        ],
    )(flat)
    return (slow_flat.reshape(C, n, H, W), fast_flat.reshape(C, T, H, W))


# speedup vs baseline: 1.6664x; 1.0010x over previous
import jax
import jax.numpy as jnp
import numpy as np
from jax.experimental import pallas as pl
from jax.experimental.pallas import tpu as pltpu

_NBUF = 6
_LAG = 3
_GMAX = 32


def _chunk_plan(nrows):
    ramp = [4, 8, 16]
    tail = [16, 8, 4]
    bulk_rows = nrows - sum(ramp) - sum(tail)
    sizes = list(ramp)
    sizes += [_GMAX] * (bulk_rows // _GMAX)
    if bulk_rows % _GMAX:
        sizes.append(bulk_rows % _GMAX)
    sizes += tail
    starts = np.cumsum([0] + sizes[:-1]).tolist()
    return list(zip(starts, sizes))


def _make_body(chunks, slow_map):
    nchunks = len(chunks)

    def body(src, fast, slow, bufs, in_sems, out_sems):
        def in_copy(g):
            b = g % _NBUF
            s0, sz = chunks[g]
            return pltpu.make_async_copy(
                src.at[pl.ds(s0, sz)], bufs.at[b].at[pl.ds(0, sz)],
                in_sems.at[b])

        def out_copies(g):
            b = g % _NBUF
            s0, sz = chunks[g]
            cps = [pltpu.make_async_copy(
                bufs.at[b].at[pl.ds(0, sz)], fast.at[pl.ds(s0, sz)],
                out_sems.at[b])]
            for k, r in slow_map[g]:
                cps.append(pltpu.make_async_copy(
                    bufs.at[b].at[r], slow.at[k], out_sems.at[b]))
            return cps

        for g in range(min(_NBUF, nchunks)):
            in_copy(g).start()
        for g in range(nchunks):
            in_copy(g).wait()
            for cp in out_copies(g):
                cp.start()
            gl = g - _LAG
            if gl >= 0:
                for cp in out_copies(gl):
                    cp.wait()
                if gl + _NBUF < nchunks:
                    in_copy(gl + _NBUF).start()
        for g in range(max(0, nchunks - _LAG), nchunks):
            for cp in out_copies(g):
                cp.wait()

    return body


def _linspace_idx(stop, num):
    div = num - 1
    step = np.arange(div, dtype=np.float32) / np.float32(div)
    out = (np.float32(0.0) * (np.float32(1.0) - step)
           + np.float32(stop) * step)
    out = np.concatenate([out, np.array([stop], dtype=np.float32)])
    return out.astype(np.int32)


def kernel(frames):
    C, T, H, W = frames.shape
    alpha = 4
    n = T // alpha
    idx = _linspace_idx(float(T - 1), n)

    nrows = C * T
    chunks = _chunk_plan(nrows)
    slow_map = {g: [] for g in range(len(chunks))}
    for c in range(C):
        for j, t in enumerate(idx.tolist()):
            r = c * T + t
            for g, (s0, sz) in enumerate(chunks):
                if s0 <= r < s0 + sz:
                    slow_map[g].append((c * n + j, r - s0))
                    break

    flat = frames.reshape(nrows, H, W)
    hbm = pl.BlockSpec(memory_space=pltpu.MemorySpace.HBM)
    fast_flat, slow_flat = pl.pallas_call(
        _make_body(chunks, slow_map),
        in_specs=[hbm],
        out_specs=[hbm, hbm],
        out_shape=[
            jax.ShapeDtypeStruct((nrows, H, W), jnp.float32),
            jax.ShapeDtypeStruct((C * n, H, W), jnp.float32),
        ],
        scratch_shapes=[
            pltpu.VMEM((_NBUF, _GMAX, H, W), jnp.float32),
            pltpu.SemaphoreType.DMA((_NBUF,)),
            pltpu.SemaphoreType.DMA((_NBUF,)),
        ],
    )(flat)
    return (slow_flat.reshape(C, n, H, W), fast_flat.reshape(C, T, H, W))
